# Initial kernel scaffold; baseline (speedup 1.0000x reference)
import jax
import jax.numpy as jnp
from jax import lax
from jax.experimental import pallas as pl
from jax.experimental.pallas import tpu as pltpu
from jax.experimental.pallas import tpu_sc as plsc

N = 10000
E = 320000
P = 10000
D = 128

NC = 2
NS = 16
NT = NC * NS
CH = 128
K = 80
EP = NT * K * CH
NACC = 10016
RPT = NACC // NS

PP = 12288
PK = (2 * PP) // (NT * CH)

_mesh = plsc.VectorSubcoreMesh(core_axis_name="c", subcore_axis_name="s")


def _sc_agg(h, srcp, dstp, ewp, with_deg):
  out_type = [jax.ShapeDtypeStruct((NC, NACC, D), jnp.float32)]
  scratch = [
      pltpu.VMEM((K, CH), jnp.int32),
      pltpu.VMEM((K, CH), jnp.int32),
      pltpu.VMEM((K, CH), jnp.float32),
      pltpu.VMEM((2, CH, D), jnp.float32),
      pltpu.VMEM_SHARED((NACC, D), jnp.float32),
      pltpu.SemaphoreType.DMA,
      pltpu.SemaphoreType.DMA,
  ]
  if with_deg:
    out_type.append(jax.ShapeDtypeStruct((NC, NACC, 16), jnp.float32))
    scratch += [
        pltpu.VMEM((CH, 16), jnp.float32),
        pltpu.VMEM((RPT, 16), jnp.float32),
        pltpu.VMEM_SHARED((NACC, 16), jnp.float32),
    ]

  def body(*refs):
    if with_deg:
      (h_hbm, src_hbm, dst_hbm, ew_hbm, agg_out, deg_out,
       src_v, dst_v, ew_v, rows_v, acc_s, sem0, sem1,
       ones_v, zcol_v, deg_s) = refs
    else:
      (h_hbm, src_hbm, dst_hbm, ew_hbm, agg_out,
       src_v, dst_v, ew_v, rows_v, acc_s, sem0, sem1) = refs

    c = lax.axis_index("c")
    s = lax.axis_index("s")
    w = s * NC + c
    base_chunk = w * K
    row0 = s * RPT

    pltpu.sync_copy(src_hbm.at[pl.ds(base_chunk, K)], src_v)
    pltpu.sync_copy(dst_hbm.at[pl.ds(base_chunk, K)], dst_v)
    pltpu.sync_copy(ew_hbm.at[pl.ds(base_chunk, K)], ew_v)

    zbuf = rows_v.at[0]
    @pl.loop(0, CH)
    def _(r):
      for cp in range(D // 16):
        zbuf[r, pl.ds(cp * 16, 16)] = jnp.zeros((16,), jnp.float32)
    for t in range(RPT // CH):
      pltpu.sync_copy(zbuf, acc_s.at[pl.ds(row0 + t * CH, CH)])
    rem = RPT % CH
    if rem:
      pltpu.sync_copy(zbuf.at[pl.ds(0, rem)],
                      acc_s.at[pl.ds(row0 + (RPT // CH) * CH, rem)])

    if with_deg:
      @pl.loop(0, CH)
      def _(r):
        ones_v[r, pl.ds(0, 16)] = jnp.ones((16,), jnp.float32)
      @pl.loop(0, RPT)
      def _(r):
        zcol_v[r, pl.ds(0, 16)] = jnp.zeros((16,), jnp.float32)
      pltpu.sync_copy(zcol_v, deg_s.at[pl.ds(row0, RPT)])

    pltpu.async_copy(h_hbm.at[src_v.at[0]], rows_v.at[0], sem0)
    plsc.subcore_barrier()

    def scale(g, b):
      buf = rows_v.at[b]
      @pl.loop(0, CH)
      def _(e):
        gi = jnp.full((16,), g, jnp.int32)
        ei = jnp.full((16,), e, jnp.int32)
        we = plsc.load_gather(ew_v, [gi, ei])
        for cp in range(D // 16):
          sl = pl.ds(cp * 16, 16)
          buf[e, sl] = buf[e, sl] * we

    @pl.loop(0, K, step=2)
    def _(g):
      pltpu.make_async_copy(h_hbm.at[src_v.at[g]], rows_v.at[0], sem0).wait()
      d1 = pltpu.async_copy(h_hbm.at[src_v.at[g + 1]], rows_v.at[1], sem1)
      scale(g, 0)
      pltpu.sync_copy(rows_v.at[0], acc_s.at[dst_v.at[g]], add=True)
      if with_deg:
        pltpu.sync_copy(ones_v, deg_s.at[dst_v.at[g]], add=True)
      d1.wait()
      @pl.when(g + 2 < K)
      def _():
        pltpu.async_copy(h_hbm.at[src_v.at[g + 2]], rows_v.at[0], sem0)
      scale(g + 1, 1)
      pltpu.sync_copy(rows_v.at[1], acc_s.at[dst_v.at[g + 1]], add=True)
      if with_deg:
        pltpu.sync_copy(ones_v, deg_s.at[dst_v.at[g + 1]], add=True)

    plsc.subcore_barrier()
    pltpu.sync_copy(acc_s.at[pl.ds(row0, RPT)], agg_out.at[c, pl.ds(row0, RPT)])
    if with_deg:
      pltpu.sync_copy(deg_s.at[pl.ds(row0, RPT)], deg_out.at[c, pl.ds(row0, RPT)])

  return pl.kernel(body, out_type=out_type, mesh=_mesh,
                   scratch_types=scratch)(h, srcp, dstp, ewp)


def _sc_pairs(h, sidx, didx):
  out_type = jax.ShapeDtypeStruct((2 * PP, D), jnp.float32)
  scratch = [
      pltpu.VMEM((PK, CH), jnp.int32),
      pltpu.VMEM((PK, CH), jnp.int32),
      pltpu.VMEM((CH, D), jnp.float32),
      pltpu.VMEM((CH, D), jnp.float32),
      pltpu.SemaphoreType.DMA,
      pltpu.SemaphoreType.DMA,
  ]

  def body(h_hbm, s_hbm, d_hbm, z_out, s_v, d_v, hs_v, hd_v, sem0, sem1):
    c = lax.axis_index("c")
    s = lax.axis_index("s")
    w = s * NC + c
    pltpu.sync_copy(s_hbm.at[pl.ds(w * PK, PK)], s_v)
    pltpu.sync_copy(d_hbm.at[pl.ds(w * PK, PK)], d_v)
    for k in range(PK):
      da = pltpu.async_copy(h_hbm.at[s_v.at[k]], hs_v, sem0)
      db = pltpu.async_copy(h_hbm.at[d_v.at[k]], hd_v, sem1)
      da.wait()
      db.wait()
      @pl.loop(0, CH)
      def _(r):
        for cp in range(D // 16):
          sl = pl.ds(cp * 16, 16)
          hs_v[r, sl] = hs_v[r, sl] * hd_v[r, sl]
      pltpu.sync_copy(hs_v, z_out.at[pl.ds((w * PK + k) * CH, CH)])

  return pl.kernel(body, out_type=out_type, mesh=_mesh,
                   scratch_types=scratch)(h, sidx, didx)


def _tc_combine(aggp, degp, h, Wn, Ws, b, relu):
  BN = 2000

  def body(a_ref, d_ref, h_ref, wn_ref, ws_ref, b_ref, o_ref):
    agg = a_ref[0] + a_ref[1]
    deg = jnp.maximum(d_ref[0, :, :1] + d_ref[1, :, :1], 1.0)
    neigh = agg / deg
    out = (jnp.dot(neigh, wn_ref[...], preferred_element_type=jnp.float32)
           + jnp.dot(h_ref[...], ws_ref[...], preferred_element_type=jnp.float32)
           + b_ref[...])
    if relu:
      out = jnp.maximum(out, 0.0)
    o_ref[...] = out

  return pl.pallas_call(
      body,
      grid=(N // BN,),
      in_specs=[
          pl.BlockSpec((NC, BN, D), lambda i: (0, i, 0)),
          pl.BlockSpec((NC, BN, 16), lambda i: (0, i, 0)),
          pl.BlockSpec((BN, D), lambda i: (i, 0)),
          pl.BlockSpec((D, D), lambda i: (0, 0)),
          pl.BlockSpec((D, D), lambda i: (0, 0)),
          pl.BlockSpec((1, D), lambda i: (0, 0)),
      ],
      out_specs=pl.BlockSpec((BN, D), lambda i: (i, 0)),
      out_shape=jax.ShapeDtypeStruct((N, D), jnp.float32),
  )(aggp, degp, h, Wn, Ws, b.reshape(1, D))


def _tc_mlp(z, PW1, pb1, PW2, pb2, PW3, pb3):
  BN = 2048
  R = z.shape[0]

  def body(z_ref, w1_ref, b1_ref, w2_ref, b2_ref, w3_ref, b3_ref, o_ref):
    a = jnp.maximum(
        jnp.dot(z_ref[...], w1_ref[...], preferred_element_type=jnp.float32)
        + b1_ref[...], 0.0)
    a = jnp.maximum(
        jnp.dot(a, w2_ref[...], preferred_element_type=jnp.float32)
        + b2_ref[...], 0.0)
    o_ref[...] = (jnp.dot(a, w3_ref[...], preferred_element_type=jnp.float32)
                  + b3_ref[...])

  return pl.pallas_call(
      body,
      grid=(R // BN,),
      in_specs=[
          pl.BlockSpec((BN, D), lambda i: (i, 0)),
          pl.BlockSpec((D, D), lambda i: (0, 0)),
          pl.BlockSpec((1, D), lambda i: (0, 0)),
          pl.BlockSpec((D, D), lambda i: (0, 0)),
          pl.BlockSpec((1, D), lambda i: (0, 0)),
          pl.BlockSpec((D, 1), lambda i: (0, 0)),
          pl.BlockSpec((1, 1), lambda i: (0, 0)),
      ],
      out_specs=pl.BlockSpec((BN, 1), lambda i: (i, 0)),
      out_shape=jax.ShapeDtypeStruct((R, 1), jnp.float32),
  )(z, PW1, pb1.reshape(1, D), PW2, pb2.reshape(1, D), PW3, pb3.reshape(1, 1))


def _chunk_edges(src, dst, ew):
  pad = EP - E
  srcp = jnp.concatenate([src, jnp.zeros((pad,), jnp.int32)])
  dstp = jnp.concatenate([dst, jnp.full((pad,), N, jnp.int32)])
  ewp = jnp.concatenate([ew, jnp.zeros((pad,), jnp.float32)])

  def arrange(a):
    return a.reshape(K, NT, CH).transpose(1, 0, 2).reshape(NT * K, CH)

  return arrange(srcp), arrange(dstp), arrange(ewp)


def _pad_pairs(a, b):
  pad = PP - P
  ap = jnp.concatenate([a, jnp.zeros((pad,), jnp.int32)])
  bp = jnp.concatenate([b, jnp.zeros((pad,), jnp.int32)])
  return ap, bp


def kernel(x, edge_index, edge_weight, pos_src, pos_dst, neg_src, neg_dst,
           Wn0, Ws0, b0, Wn1, Ws1, b1, Wn2, Ws2, b2,
           PW1, pb1, PW2, pb2, PW3, pb3):
  src = edge_index[0]
  dst = edge_index[1]
  srcp, dstp, ewp = _chunk_edges(src, dst, edge_weight)

  agg0, deg = _sc_agg(x, srcp, dstp, ewp, with_deg=True)
  h1 = _tc_combine(agg0, deg, x, Wn0, Ws0, b0, relu=True)
  (agg1,) = _sc_agg(h1, srcp, dstp, ewp, with_deg=False)
  h2 = _tc_combine(agg1, deg, h1, Wn1, Ws1, b1, relu=True)
  (agg2,) = _sc_agg(h2, srcp, dstp, ewp, with_deg=False)
  h3 = _tc_combine(agg2, deg, h2, Wn2, Ws2, b2, relu=False)

  ps, pd = _pad_pairs(pos_src, pos_dst)
  ns, nd = _pad_pairs(neg_src, neg_dst)
  sidx = jnp.concatenate([ps, ns]).reshape(NT * PK, CH)
  didx = jnp.concatenate([pd, nd]).reshape(NT * PK, CH)
  z = _sc_pairs(h3, sidx, didx)
  scores = _tc_mlp(z, PW1, pb1, PW2, pb2, PW3, pb3)
  h_pos = scores[:P]
  h_neg = scores[PP:PP + P]
  return (h_pos, h_neg, h3)


# speedup vs baseline: 2.7822x; 2.7822x over previous
import dataclasses

import jax
import jax.numpy as jnp
from jax import lax
from jax.experimental import pallas as pl
from jax.experimental.pallas import tpu as pltpu
from jax.experimental.pallas import tpu_sc as plsc

N = 10000
E = 320000
P = 10000
D = 128

NC = 2
NS = 16
NT = NC * NS
CH = 64
K = 160
EP = NT * K * CH
NACC = 10112
RPT = NACC // NS

PP = 10240
PK = (2 * PP) // (NT * CH)

_mesh = plsc.VectorSubcoreMesh(core_axis_name="c", subcore_axis_name="s",
                               num_cores=NC, num_subcores=NS)

_sc_params = pltpu.CompilerParams()
if "needs_layout_passes" in pltpu.CompilerParams.__dataclass_fields__:
  _sc_params = dataclasses.replace(_sc_params, needs_layout_passes=False)


def _sc_agg(h, srcs, dsts, ww, zacc):
  out_type = jax.ShapeDtypeStruct((NC, NACC, D), jnp.float32)
  scratch = [
      pltpu.VMEM((CH,), jnp.int32),
      pltpu.VMEM((CH,), jnp.int32),
      pltpu.VMEM((CH,), jnp.int32),
      pltpu.VMEM((CH,), jnp.int32),
      pltpu.VMEM((CH, 16), jnp.float32),
      pltpu.VMEM((CH, 16), jnp.float32),
      pltpu.VMEM((2, CH, D), jnp.float32),
      pltpu.VMEM_SHARED((NACC, D), jnp.float32),
      pltpu.SemaphoreType.DMA,
      pltpu.SemaphoreType.DMA,
      pltpu.SemaphoreType.DMA,
      pltpu.SemaphoreType.DMA,
  ]

  def body(h_hbm, s_hbm, d_hbm, w_hbm, zacc_hbm, agg_out,
           si0, si1, di0, di1, wwv0, wwv1, rows_v, acc_s, g0, g1, m0, m1):
    c = lax.axis_index("c")
    s = lax.axis_index("s")
    w = s * NC + c
    base = w * K
    row0 = s * RPT
    si = [si0, si1]
    di = [di0, di1]
    wwv = [wwv0, wwv1]
    gsem = [g0, g1]
    msem = [m0, m1]

    def meta_copy(g, b):
      pltpu.async_copy(s_hbm.at[base + g], si[b], msem[b])
      pltpu.async_copy(d_hbm.at[base + g], di[b], msem[b])
      pltpu.async_copy(w_hbm.at[base + g], wwv[b], msem[b])

    def meta_wait(g, b):
      pltpu.make_async_copy(s_hbm.at[base + g], si[b], msem[b]).wait()
      pltpu.make_async_copy(d_hbm.at[base + g], di[b], msem[b]).wait()
      pltpu.make_async_copy(w_hbm.at[base + g], wwv[b], msem[b]).wait()

    def gather(g, b):
      return pltpu.async_copy(h_hbm.at[si[b]], rows_v.at[b], gsem[b])

    def scale(b):
      buf = rows_v.at[b]
      wbuf = wwv[b]
      @pl.loop(0, CH)
      def _(e):
        we = wbuf[e, pl.ds(0, 16)]
        for cp in range(D // 16):
          sl = pl.ds(cp * 16, 16)
          buf[e, sl] = buf[e, sl] * we

    def scatter(b):
      pltpu.sync_copy(rows_v.at[b], acc_s.at[di[b]], add=True)

    pltpu.sync_copy(zacc_hbm.at[pl.ds(row0, RPT)], acc_s.at[pl.ds(row0, RPT)])

    meta_copy(0, 0)
    meta_copy(1, 1)
    plsc.subcore_barrier()

    @pl.loop(0, K, step=2)
    def _(g):
      meta_wait(g, 0)
      ga = gather(g, 0)
      meta_wait(g + 1, 1)
      gb = gather(g + 1, 1)
      ga.wait()
      scale(0)
      scatter(0)
      @pl.when(g + 2 < K)
      def _():
        meta_copy(g + 2, 0)
      gb.wait()
      scale(1)
      scatter(1)
      @pl.when(g + 3 < K)
      def _():
        meta_copy(g + 3, 1)

    plsc.subcore_barrier()
    pltpu.sync_copy(acc_s.at[pl.ds(row0, RPT)], agg_out.at[c, pl.ds(row0, RPT)])

  return pl.kernel(body, out_type=out_type, mesh=_mesh,
                   compiler_params=_sc_params,
                   scratch_types=scratch)(h, srcs, dsts, ww, zacc)


def _sc_deg(dsts, zacc):
  out_type = jax.ShapeDtypeStruct((NC, NACC, D), jnp.float32)
  scratch = [
      pltpu.VMEM((CH,), jnp.int32),
      pltpu.VMEM((CH,), jnp.int32),
      pltpu.VMEM((CH, D), jnp.float32),
      pltpu.VMEM_SHARED((NACC, D), jnp.float32),
      pltpu.SemaphoreType.DMA,
      pltpu.SemaphoreType.DMA,
      pltpu.SemaphoreType.DMA,
  ]

  def body(d_hbm, zdeg_hbm, deg_out, b0, b1, ones_v, deg_s, isem, ssem0,
           ssem1):
    c = lax.axis_index("c")
    s = lax.axis_index("s")
    w = s * NC + c
    base = w * K
    row0 = s * RPT
    pltpu.sync_copy(zdeg_hbm.at[pl.ds(row0, RPT)], deg_s.at[pl.ds(row0, RPT)])
    @pl.loop(0, CH)
    def _(r):
      for cp in range(D // 16):
        ones_v[r, pl.ds(cp * 16, 16)] = jnp.ones((16,), jnp.float32)
    pltpu.sync_copy(d_hbm.at[base], b0)
    plsc.subcore_barrier()
    @pl.loop(0, K, step=2)
    def _(g):
      pltpu.async_copy(d_hbm.at[base + g + 1], b1, isem)
      d0 = pltpu.async_copy(ones_v, deg_s.at[b0], ssem0, add=True)
      pltpu.make_async_copy(d_hbm.at[base + g + 1], b1, isem).wait()
      d1 = pltpu.async_copy(ones_v, deg_s.at[b1], ssem1, add=True)
      d0.wait()
      d1.wait()
      @pl.when(g + 2 < K)
      def _():
        pltpu.sync_copy(d_hbm.at[base + g + 2], b0)
    plsc.subcore_barrier()
    pltpu.sync_copy(deg_s.at[pl.ds(row0, RPT)], deg_out.at[c, pl.ds(row0, RPT)])

  return pl.kernel(body, out_type=out_type, mesh=_mesh,
                   compiler_params=_sc_params,
                   scratch_types=scratch)(dsts, zacc)


def _sc_pairs(h, sidx, didx):
  out_type = jax.ShapeDtypeStruct((2 * PP, D), jnp.float32)
  scratch = [
      pltpu.VMEM((PK, CH), jnp.int32),
      pltpu.VMEM((PK, CH), jnp.int32),
      pltpu.VMEM((CH, D), jnp.float32),
      pltpu.VMEM((CH, D), jnp.float32),
      pltpu.SemaphoreType.DMA,
      pltpu.SemaphoreType.DMA,
  ]

  def body(h_hbm, s_hbm, d_hbm, z_out, s_v, d_v, hs_v, hd_v, sem0, sem1):
    c = lax.axis_index("c")
    s = lax.axis_index("s")
    w = s * NC + c
    pltpu.sync_copy(s_hbm.at[w], s_v)
    pltpu.sync_copy(d_hbm.at[w], d_v)
    for k in range(PK):
      da = pltpu.async_copy(h_hbm.at[s_v.at[k]], hs_v, sem0)
      db = pltpu.async_copy(h_hbm.at[d_v.at[k]], hd_v, sem1)
      da.wait()
      db.wait()
      @pl.loop(0, CH)
      def _(r):
        for cp in range(D // 16):
          sl = pl.ds(cp * 16, 16)
          hs_v[r, sl] = hs_v[r, sl] * hd_v[r, sl]
      pltpu.sync_copy(hs_v, z_out.at[pl.ds((w * PK + k) * CH, CH)])

  return pl.kernel(body, out_type=out_type, mesh=_mesh,
                   compiler_params=_sc_params,
                   scratch_types=scratch)(h, sidx, didx)


def _tc_combine(aggp, degp, h, Wn, Ws, b, relu):
  BN = 2000

  def body(a_ref, d_ref, h_ref, wn_ref, ws_ref, b_ref, o_ref):
    agg = a_ref[0] + a_ref[1]
    deg = jnp.maximum(d_ref[0, :, :1] + d_ref[1, :, :1], 1.0)
    neigh = agg / deg
    out = (jnp.dot(neigh, wn_ref[...], preferred_element_type=jnp.float32)
           + jnp.dot(h_ref[...], ws_ref[...], preferred_element_type=jnp.float32)
           + b_ref[...])
    if relu:
      out = jnp.maximum(out, 0.0)
    o_ref[...] = out

  return pl.pallas_call(
      body,
      grid=(N // BN,),
      in_specs=[
          pl.BlockSpec((NC, BN, D), lambda i: (0, i, 0)),
          pl.BlockSpec((NC, BN, D), lambda i: (0, i, 0)),
          pl.BlockSpec((BN, D), lambda i: (i, 0)),
          pl.BlockSpec((D, D), lambda i: (0, 0)),
          pl.BlockSpec((D, D), lambda i: (0, 0)),
          pl.BlockSpec((1, D), lambda i: (0, 0)),
      ],
      out_specs=pl.BlockSpec((BN, D), lambda i: (i, 0)),
      out_shape=jax.ShapeDtypeStruct((N, D), jnp.float32),
  )(aggp, degp, h, Wn, Ws, b.reshape(1, D))


def _tc_mlp(z, PW1, pb1, PW2, pb2, PW3, pb3):
  BN = 2048
  R = z.shape[0]

  def body(z_ref, w1_ref, b1_ref, w2_ref, b2_ref, w3_ref, b3_ref, o_ref):
    a = jnp.maximum(
        jnp.dot(z_ref[...], w1_ref[...], preferred_element_type=jnp.float32)
        + b1_ref[...], 0.0)
    a = jnp.maximum(
        jnp.dot(a, w2_ref[...], preferred_element_type=jnp.float32)
        + b2_ref[...], 0.0)
    o_ref[...] = (jnp.dot(a, w3_ref[...], preferred_element_type=jnp.float32)
                  + b3_ref[...])

  return pl.pallas_call(
      body,
      grid=(R // BN,),
      in_specs=[
          pl.BlockSpec((BN, D), lambda i: (i, 0)),
          pl.BlockSpec((D, D), lambda i: (0, 0)),
          pl.BlockSpec((1, D), lambda i: (0, 0)),
          pl.BlockSpec((D, D), lambda i: (0, 0)),
          pl.BlockSpec((1, D), lambda i: (0, 0)),
          pl.BlockSpec((D, 1), lambda i: (0, 0)),
          pl.BlockSpec((1, 1), lambda i: (0, 0)),
      ],
      out_specs=pl.BlockSpec((BN, 1), lambda i: (i, 0)),
      out_shape=jax.ShapeDtypeStruct((R, 1), jnp.float32),
  )(z, PW1, pb1.reshape(1, D), PW2, pb2.reshape(1, D), PW3, pb3.reshape(1, 1))


def _chunk_edges(src, dst, ew):
  pad = EP - E
  srcp = jnp.concatenate([src, jnp.zeros((pad,), jnp.int32)])
  dstp = jnp.concatenate([dst, jnp.full((pad,), N, jnp.int32)])
  ewp = jnp.concatenate([ew, jnp.zeros((pad,), jnp.float32)])

  def arrange(a):
    return a.reshape(K, NT, CH).transpose(1, 0, 2).reshape(NT * K, CH)

  srcs = arrange(srcp)
  dsts = arrange(dstp)
  ww = jnp.broadcast_to(arrange(ewp)[:, :, None], (NT * K, CH, 16))
  return srcs, dsts, ww


def _pad_pairs(a, b):
  pad = PP - P
  ap = jnp.concatenate([a, jnp.zeros((pad,), jnp.int32)])
  bp = jnp.concatenate([b, jnp.zeros((pad,), jnp.int32)])
  return ap, bp


def kernel(x, edge_index, edge_weight, pos_src, pos_dst, neg_src, neg_dst,
           Wn0, Ws0, b0, Wn1, Ws1, b1, Wn2, Ws2, b2,
           PW1, pb1, PW2, pb2, PW3, pb3):
  src = edge_index[0]
  dst = edge_index[1]
  srcs, dsts, ww = _chunk_edges(src, dst, edge_weight)
  zacc = jnp.zeros((NACC, D), jnp.float32)

  deg = _sc_deg(dsts, zacc)
  agg0 = _sc_agg(x, srcs, dsts, ww, zacc)
  h1 = _tc_combine(agg0, deg, x, Wn0, Ws0, b0, relu=True)
  agg1 = _sc_agg(h1, srcs, dsts, ww, zacc)
  h2 = _tc_combine(agg1, deg, h1, Wn1, Ws1, b1, relu=True)
  agg2 = _sc_agg(h2, srcs, dsts, ww, zacc)
  h3 = _tc_combine(agg2, deg, h2, Wn2, Ws2, b2, relu=False)

  ps, pd = _pad_pairs(pos_src, pos_dst)
  ns, nd = _pad_pairs(neg_src, neg_dst)
  sidx = jnp.concatenate([ps, ns]).reshape(NT, PK, CH)
  didx = jnp.concatenate([pd, nd]).reshape(NT, PK, CH)
  z = _sc_pairs(h3, sidx, didx)
  scores = _tc_mlp(z, PW1, pb1, PW2, pb2, PW3, pb3)
  h_pos = scores[:P]
  h_neg = scores[PP:PP + P]
  return (h_pos, h_neg, h3)


# speedup vs baseline: 3.4518x; 1.2407x over previous
import dataclasses

import jax
import jax.numpy as jnp
from jax import lax
from jax.experimental import pallas as pl
from jax.experimental.pallas import tpu as pltpu
from jax.experimental.pallas import tpu_sc as plsc

N = 10000
E = 320000
P = 10000
D = 128

NC = 2
NS = 16
NT = NC * NS
CH = 80
K = 126
EP = NT * K * CH
NACC = 10112
RPT = NACC // NS

PP = 10240
PK = (2 * PP) // (NT * CH)

_mesh = plsc.VectorSubcoreMesh(core_axis_name="c", subcore_axis_name="s",
                               num_cores=NC, num_subcores=NS)

_sc_params = pltpu.CompilerParams()
if "needs_layout_passes" in pltpu.CompilerParams.__dataclass_fields__:
  _sc_params = dataclasses.replace(_sc_params, needs_layout_passes=False)


def _sc_agg(h, srcs, dsts, ww, zacc):
  out_type = jax.ShapeDtypeStruct((NC, NACC, D), jnp.float32)
  scratch = [
      pltpu.VMEM((CH,), jnp.int32),
      pltpu.VMEM((CH,), jnp.int32),
      pltpu.VMEM((CH,), jnp.int32),
      pltpu.VMEM((CH,), jnp.int32),
      pltpu.VMEM((CH, 16), jnp.float32),
      pltpu.VMEM((CH, 16), jnp.float32),
      pltpu.VMEM((2, CH, D), jnp.float32),
      pltpu.VMEM_SHARED((NACC, D), jnp.float32),
      pltpu.SemaphoreType.DMA,
      pltpu.SemaphoreType.DMA,
      pltpu.SemaphoreType.DMA,
      pltpu.SemaphoreType.DMA,
  ]

  def body(h_hbm, s_hbm, d_hbm, w_hbm, zacc_hbm, agg_out,
           si0, si1, di0, di1, wwv0, wwv1, rows_v, acc_s, g0, g1, m0, m1):
    c = lax.axis_index("c")
    s = lax.axis_index("s")
    w = s * NC + c
    base = w * K
    row0 = s * RPT
    si = [si0, si1]
    di = [di0, di1]
    wwv = [wwv0, wwv1]
    gsem = [g0, g1]
    msem = [m0, m1]

    def meta_copy(g, b):
      pltpu.async_copy(s_hbm.at[base + g], si[b], msem[b])
      pltpu.async_copy(d_hbm.at[base + g], di[b], msem[b])
      pltpu.async_copy(w_hbm.at[base + g], wwv[b], msem[b])

    def meta_wait(g, b):
      pltpu.make_async_copy(s_hbm.at[base + g], si[b], msem[b]).wait()
      pltpu.make_async_copy(d_hbm.at[base + g], di[b], msem[b]).wait()
      pltpu.make_async_copy(w_hbm.at[base + g], wwv[b], msem[b]).wait()

    def gather(g, b):
      return pltpu.async_copy(h_hbm.at[si[b]], rows_v.at[b], gsem[b])

    def scale(b):
      buf = rows_v.at[b]
      wbuf = wwv[b]
      @pl.loop(0, CH)
      def _(e):
        we = wbuf[e, pl.ds(0, 16)]
        for cp in range(D // 16):
          sl = pl.ds(cp * 16, 16)
          buf[e, sl] = buf[e, sl] * we

    def scatter(b, ssem):
      return pltpu.async_copy(rows_v.at[b], acc_s.at[di[b]], ssem, add=True)

    pltpu.sync_copy(zacc_hbm.at[pl.ds(row0, RPT)], acc_s.at[pl.ds(row0, RPT)])

    meta_copy(0, 0)
    meta_copy(1, 1)
    meta_wait(0, 0)
    meta_wait(1, 1)
    plsc.subcore_barrier()

    @pl.loop(0, K, step=2)
    def _(g):
      ga = gather(g, 0)
      gb = gather(g + 1, 1)
      ga.wait()
      scale(0)
      sa = scatter(0, gsem[0])
      gb.wait()
      scale(1)
      sb = scatter(1, gsem[1])
      sa.wait()
      sb.wait()
      @pl.when(g + 2 < K)
      def _():
        meta_copy(g + 2, 0)
        meta_wait(g + 2, 0)
      @pl.when(g + 3 < K)
      def _():
        meta_copy(g + 3, 1)
        meta_wait(g + 3, 1)

    plsc.subcore_barrier()
    pltpu.sync_copy(acc_s.at[pl.ds(row0, RPT)], agg_out.at[c, pl.ds(row0, RPT)])

  return pl.kernel(body, out_type=out_type, mesh=_mesh,
                   compiler_params=_sc_params,
                   scratch_types=scratch)(h, srcs, dsts, ww, zacc)


def _sc_deg(dsts, zacc):
  out_type = jax.ShapeDtypeStruct((NC, NACC, D), jnp.float32)
  scratch = [
      pltpu.VMEM((CH,), jnp.int32),
      pltpu.VMEM((CH,), jnp.int32),
      pltpu.VMEM((CH, D), jnp.float32),
      pltpu.VMEM_SHARED((NACC, D), jnp.float32),
      pltpu.SemaphoreType.DMA,
      pltpu.SemaphoreType.DMA,
      pltpu.SemaphoreType.DMA,
  ]

  def body(d_hbm, zdeg_hbm, deg_out, b0, b1, ones_v, deg_s, isem, ssem0,
           ssem1):
    c = lax.axis_index("c")
    s = lax.axis_index("s")
    w = s * NC + c
    base = w * K
    row0 = s * RPT
    pltpu.sync_copy(zdeg_hbm.at[pl.ds(row0, RPT)], deg_s.at[pl.ds(row0, RPT)])
    @pl.loop(0, CH)
    def _(r):
      for cp in range(D // 16):
        ones_v[r, pl.ds(cp * 16, 16)] = jnp.ones((16,), jnp.float32)
    pltpu.sync_copy(d_hbm.at[base], b0)
    plsc.subcore_barrier()
    @pl.loop(0, K, step=2)
    def _(g):
      pltpu.async_copy(d_hbm.at[base + g + 1], b1, isem)
      d0 = pltpu.async_copy(ones_v, deg_s.at[b0], ssem0, add=True)
      pltpu.make_async_copy(d_hbm.at[base + g + 1], b1, isem).wait()
      d1 = pltpu.async_copy(ones_v, deg_s.at[b1], ssem1, add=True)
      d0.wait()
      d1.wait()
      @pl.when(g + 2 < K)
      def _():
        pltpu.sync_copy(d_hbm.at[base + g + 2], b0)
    plsc.subcore_barrier()
    pltpu.sync_copy(deg_s.at[pl.ds(row0, RPT)], deg_out.at[c, pl.ds(row0, RPT)])

  return pl.kernel(body, out_type=out_type, mesh=_mesh,
                   compiler_params=_sc_params,
                   scratch_types=scratch)(dsts, zacc)


def _sc_pairs(h, sidx, didx):
  out_type = jax.ShapeDtypeStruct((2 * PP, D), jnp.float32)
  scratch = [
      pltpu.VMEM((PK, CH), jnp.int32),
      pltpu.VMEM((PK, CH), jnp.int32),
      pltpu.VMEM((CH, D), jnp.float32),
      pltpu.VMEM((CH, D), jnp.float32),
      pltpu.SemaphoreType.DMA,
      pltpu.SemaphoreType.DMA,
  ]

  def body(h_hbm, s_hbm, d_hbm, z_out, s_v, d_v, hs_v, hd_v, sem0, sem1):
    c = lax.axis_index("c")
    s = lax.axis_index("s")
    w = s * NC + c
    pltpu.sync_copy(s_hbm.at[w], s_v)
    pltpu.sync_copy(d_hbm.at[w], d_v)
    for k in range(PK):
      da = pltpu.async_copy(h_hbm.at[s_v.at[k]], hs_v, sem0)
      db = pltpu.async_copy(h_hbm.at[d_v.at[k]], hd_v, sem1)
      da.wait()
      db.wait()
      @pl.loop(0, CH)
      def _(r):
        for cp in range(D // 16):
          sl = pl.ds(cp * 16, 16)
          hs_v[r, sl] = hs_v[r, sl] * hd_v[r, sl]
      pltpu.sync_copy(hs_v, z_out.at[pl.ds((w * PK + k) * CH, CH)])

  return pl.kernel(body, out_type=out_type, mesh=_mesh,
                   compiler_params=_sc_params,
                   scratch_types=scratch)(h, sidx, didx)


def _tc_combine(aggp, degp, h, Wn, Ws, b, relu):
  BN = 2000

  def body(a_ref, d_ref, h_ref, wn_ref, ws_ref, b_ref, o_ref):
    agg = a_ref[0] + a_ref[1]
    deg = jnp.maximum(d_ref[0, :, :1] + d_ref[1, :, :1], 1.0)
    neigh = agg / deg
    out = (jnp.dot(neigh, wn_ref[...], preferred_element_type=jnp.float32)
           + jnp.dot(h_ref[...], ws_ref[...], preferred_element_type=jnp.float32)
           + b_ref[...])
    if relu:
      out = jnp.maximum(out, 0.0)
    o_ref[...] = out

  return pl.pallas_call(
      body,
      grid=(N // BN,),
      in_specs=[
          pl.BlockSpec((NC, BN, D), lambda i: (0, i, 0)),
          pl.BlockSpec((NC, BN, D), lambda i: (0, i, 0)),
          pl.BlockSpec((BN, D), lambda i: (i, 0)),
          pl.BlockSpec((D, D), lambda i: (0, 0)),
          pl.BlockSpec((D, D), lambda i: (0, 0)),
          pl.BlockSpec((1, D), lambda i: (0, 0)),
      ],
      out_specs=pl.BlockSpec((BN, D), lambda i: (i, 0)),
      out_shape=jax.ShapeDtypeStruct((N, D), jnp.float32),
  )(aggp, degp, h, Wn, Ws, b.reshape(1, D))


def _tc_mlp(z, PW1, pb1, PW2, pb2, PW3, pb3):
  R = z.shape[0]
  BN = max(b for b in (2048, 1792, 1280, 1024, 512) if R % b == 0)

  def body(z_ref, w1_ref, b1_ref, w2_ref, b2_ref, w3_ref, b3_ref, o_ref):
    a = jnp.maximum(
        jnp.dot(z_ref[...], w1_ref[...], preferred_element_type=jnp.float32)
        + b1_ref[...], 0.0)
    a = jnp.maximum(
        jnp.dot(a, w2_ref[...], preferred_element_type=jnp.float32)
        + b2_ref[...], 0.0)
    o_ref[...] = (jnp.dot(a, w3_ref[...], preferred_element_type=jnp.float32)
                  + b3_ref[...])

  return pl.pallas_call(
      body,
      grid=(R // BN,),
      in_specs=[
          pl.BlockSpec((BN, D), lambda i: (i, 0)),
          pl.BlockSpec((D, D), lambda i: (0, 0)),
          pl.BlockSpec((1, D), lambda i: (0, 0)),
          pl.BlockSpec((D, D), lambda i: (0, 0)),
          pl.BlockSpec((1, D), lambda i: (0, 0)),
          pl.BlockSpec((D, 1), lambda i: (0, 0)),
          pl.BlockSpec((1, 1), lambda i: (0, 0)),
      ],
      out_specs=pl.BlockSpec((BN, 1), lambda i: (i, 0)),
      out_shape=jax.ShapeDtypeStruct((R, 1), jnp.float32),
  )(z, PW1, pb1.reshape(1, D), PW2, pb2.reshape(1, D), PW3, pb3.reshape(1, 1))


def _chunk_edges(src, dst, ew):
  pad = EP - E
  srcp = jnp.concatenate([src, jnp.zeros((pad,), jnp.int32)])
  dstp = jnp.concatenate([dst, jnp.full((pad,), N, jnp.int32)])
  ewp = jnp.concatenate([ew, jnp.zeros((pad,), jnp.float32)])

  def arrange(a):
    return a.reshape(K, NT, CH).transpose(1, 0, 2).reshape(NT * K, CH)

  srcs = arrange(srcp)
  dsts = arrange(dstp)
  ww = jnp.broadcast_to(arrange(ewp)[:, :, None], (NT * K, CH, 16))
  return srcs, dsts, ww


def _pad_pairs(a, b):
  pad = PP - P
  ap = jnp.concatenate([a, jnp.zeros((pad,), jnp.int32)])
  bp = jnp.concatenate([b, jnp.zeros((pad,), jnp.int32)])
  return ap, bp


def kernel(x, edge_index, edge_weight, pos_src, pos_dst, neg_src, neg_dst,
           Wn0, Ws0, b0, Wn1, Ws1, b1, Wn2, Ws2, b2,
           PW1, pb1, PW2, pb2, PW3, pb3):
  src = edge_index[0]
  dst = edge_index[1]
  srcs, dsts, ww = _chunk_edges(src, dst, edge_weight)
  zacc = jnp.zeros((NACC, D), jnp.float32)

  deg = _sc_deg(dsts, zacc)
  agg0 = _sc_agg(x, srcs, dsts, ww, zacc)
  h1 = _tc_combine(agg0, deg, x, Wn0, Ws0, b0, relu=True)
  agg1 = _sc_agg(h1, srcs, dsts, ww, zacc)
  h2 = _tc_combine(agg1, deg, h1, Wn1, Ws1, b1, relu=True)
  agg2 = _sc_agg(h2, srcs, dsts, ww, zacc)
  h3 = _tc_combine(agg2, deg, h2, Wn2, Ws2, b2, relu=False)

  ps, pd = _pad_pairs(pos_src, pos_dst)
  ns, nd = _pad_pairs(neg_src, neg_dst)
  sidx = jnp.concatenate([ps, ns]).reshape(NT, PK, CH)
  didx = jnp.concatenate([pd, nd]).reshape(NT, PK, CH)
  z = _sc_pairs(h3, sidx, didx)
  scores = _tc_mlp(z, PW1, pb1, PW2, pb2, PW3, pb3)
  h_pos = scores[:P]
  h_neg = scores[PP:PP + P]
  return (h_pos, h_neg, h3)


# speedup vs baseline: 4.6300x; 1.3413x over previous
import dataclasses

import jax
import jax.numpy as jnp
from jax import lax
from jax.experimental import pallas as pl
from jax.experimental.pallas import tpu as pltpu
from jax.experimental.pallas import tpu_sc as plsc

N = 10000
E = 320000
P = 10000
D = 128

NC = 2
NS = 16
NT = NC * NS
CH = 88
K = 114
EP = NT * K * CH
NACC = 10112
RPT = NACC // NS

PP = 11264
PK = (2 * PP) // (NT * CH)

_mesh = plsc.VectorSubcoreMesh(core_axis_name="c", subcore_axis_name="s",
                               num_cores=NC, num_subcores=NS)

_sc_params = pltpu.CompilerParams()
if "needs_layout_passes" in pltpu.CompilerParams.__dataclass_fields__:
  _sc_params = dataclasses.replace(_sc_params, needs_layout_passes=False)


def _sc_agg(h, srcs, dsts, ww, zacc):
  out_type = jax.ShapeDtypeStruct((NC, NACC, D), jnp.float32)
  scratch = [
      pltpu.VMEM((CH,), jnp.int32),
      pltpu.VMEM((CH,), jnp.int32),
      pltpu.VMEM((CH,), jnp.int32),
      pltpu.VMEM((CH,), jnp.int32),
      pltpu.VMEM((CH, 16), jnp.float32),
      pltpu.VMEM((CH, 16), jnp.float32),
      pltpu.VMEM((2, CH, D), jnp.float32),
      pltpu.VMEM_SHARED((NACC, D), jnp.float32),
      pltpu.SemaphoreType.DMA,
      pltpu.SemaphoreType.DMA,
      pltpu.SemaphoreType.DMA,
      pltpu.SemaphoreType.DMA,
  ]

  def body(h_hbm, s_hbm, d_hbm, w_hbm, zacc_hbm, agg_out,
           si0, si1, di0, di1, wwv0, wwv1, rows_v, acc_s, g0, g1, m0, m1):
    c = lax.axis_index("c")
    s = lax.axis_index("s")
    w = s * NC + c
    base = w * K
    row0 = s * RPT
    si = [si0, si1]
    di = [di0, di1]
    wwv = [wwv0, wwv1]
    gsem = [g0, g1]
    msem = [m0, m1]

    def meta_copy(g, b):
      pltpu.async_copy(s_hbm.at[base + g], si[b], msem[b])
      pltpu.async_copy(d_hbm.at[base + g], di[b], msem[b])
      pltpu.async_copy(w_hbm.at[base + g], wwv[b], msem[b])

    def meta_wait(g, b):
      pltpu.make_async_copy(s_hbm.at[base + g], si[b], msem[b]).wait()
      pltpu.make_async_copy(d_hbm.at[base + g], di[b], msem[b]).wait()
      pltpu.make_async_copy(w_hbm.at[base + g], wwv[b], msem[b]).wait()

    def gather(g, b):
      return pltpu.async_copy(h_hbm.at[si[b]], rows_v.at[b], gsem[b])

    def scale(b):
      buf = rows_v.at[b]
      wbuf = wwv[b]
      @pl.loop(0, CH)
      def _(e):
        we = wbuf[e, pl.ds(0, 16)]
        for cp in range(D // 16):
          sl = pl.ds(cp * 16, 16)
          buf[e, sl] = buf[e, sl] * we

    def scatter(b, ssem):
      return pltpu.async_copy(rows_v.at[b], acc_s.at[di[b]], ssem, add=True)

    pltpu.sync_copy(zacc_hbm.at[pl.ds(row0, RPT)], acc_s.at[pl.ds(row0, RPT)])

    meta_copy(0, 0)
    meta_copy(1, 1)
    plsc.subcore_barrier()

    @pl.loop(0, K, step=2)
    def _(g):
      meta_wait(g, 0)
      ga = gather(g, 0)
      meta_wait(g + 1, 1)
      gb = gather(g + 1, 1)
      ga.wait()
      scale(0)
      sa = scatter(0, gsem[0])
      @pl.when(g + 2 < K)
      def _():
        pltpu.async_copy(s_hbm.at[base + g + 2], si[0], msem[0])
        pltpu.async_copy(w_hbm.at[base + g + 2], wwv[0], msem[0])
      gb.wait()
      scale(1)
      sb = scatter(1, gsem[1])
      @pl.when(g + 3 < K)
      def _():
        pltpu.async_copy(s_hbm.at[base + g + 3], si[1], msem[1])
        pltpu.async_copy(w_hbm.at[base + g + 3], wwv[1], msem[1])
      sa.wait()
      @pl.when(g + 2 < K)
      def _():
        pltpu.async_copy(d_hbm.at[base + g + 2], di[0], msem[0])
      sb.wait()
      @pl.when(g + 3 < K)
      def _():
        pltpu.async_copy(d_hbm.at[base + g + 3], di[1], msem[1])

    plsc.subcore_barrier()
    pltpu.sync_copy(acc_s.at[pl.ds(row0, RPT)], agg_out.at[c, pl.ds(row0, RPT)])

  return pl.kernel(body, out_type=out_type, mesh=_mesh,
                   compiler_params=_sc_params,
                   scratch_types=scratch)(h, srcs, dsts, ww, zacc)


def _sc_deg(dsts, zacc):
  out_type = jax.ShapeDtypeStruct((NC, NACC, D), jnp.float32)
  scratch = [
      pltpu.VMEM((CH,), jnp.int32),
      pltpu.VMEM((CH,), jnp.int32),
      pltpu.VMEM((CH, D), jnp.float32),
      pltpu.VMEM_SHARED((NACC, D), jnp.float32),
      pltpu.SemaphoreType.DMA,
      pltpu.SemaphoreType.DMA,
      pltpu.SemaphoreType.DMA,
  ]

  def body(d_hbm, zdeg_hbm, deg_out, b0, b1, ones_v, deg_s, isem, ssem0,
           ssem1):
    c = lax.axis_index("c")
    s = lax.axis_index("s")
    w = s * NC + c
    base = w * K
    row0 = s * RPT
    pltpu.sync_copy(zdeg_hbm.at[pl.ds(row0, RPT)], deg_s.at[pl.ds(row0, RPT)])
    @pl.loop(0, CH)
    def _(r):
      for cp in range(D // 16):
        ones_v[r, pl.ds(cp * 16, 16)] = jnp.ones((16,), jnp.float32)
    pltpu.sync_copy(d_hbm.at[base], b0)
    plsc.subcore_barrier()
    @pl.loop(0, K, step=2)
    def _(g):
      pltpu.async_copy(d_hbm.at[base + g + 1], b1, isem)
      d0 = pltpu.async_copy(ones_v, deg_s.at[b0], ssem0, add=True)
      pltpu.make_async_copy(d_hbm.at[base + g + 1], b1, isem).wait()
      d1 = pltpu.async_copy(ones_v, deg_s.at[b1], ssem1, add=True)
      d0.wait()
      d1.wait()
      @pl.when(g + 2 < K)
      def _():
        pltpu.sync_copy(d_hbm.at[base + g + 2], b0)
    plsc.subcore_barrier()
    pltpu.sync_copy(deg_s.at[pl.ds(row0, RPT)], deg_out.at[c, pl.ds(row0, RPT)])

  return pl.kernel(body, out_type=out_type, mesh=_mesh,
                   compiler_params=_sc_params,
                   scratch_types=scratch)(dsts, zacc)


def _sc_pairs(h, sidx, didx):
  out_type = jax.ShapeDtypeStruct((2 * PP, D), jnp.float32)
  scratch = [
      pltpu.VMEM((PK, CH), jnp.int32),
      pltpu.VMEM((PK, CH), jnp.int32),
      pltpu.VMEM((CH, D), jnp.float32),
      pltpu.VMEM((CH, D), jnp.float32),
      pltpu.SemaphoreType.DMA,
      pltpu.SemaphoreType.DMA,
  ]

  def body(h_hbm, s_hbm, d_hbm, z_out, s_v, d_v, hs_v, hd_v, sem0, sem1):
    c = lax.axis_index("c")
    s = lax.axis_index("s")
    w = s * NC + c
    pltpu.sync_copy(s_hbm.at[w], s_v)
    pltpu.sync_copy(d_hbm.at[w], d_v)
    for k in range(PK):
      da = pltpu.async_copy(h_hbm.at[s_v.at[k]], hs_v, sem0)
      db = pltpu.async_copy(h_hbm.at[d_v.at[k]], hd_v, sem1)
      da.wait()
      db.wait()
      @pl.loop(0, CH)
      def _(r):
        for cp in range(D // 16):
          sl = pl.ds(cp * 16, 16)
          hs_v[r, sl] = hs_v[r, sl] * hd_v[r, sl]
      pltpu.sync_copy(hs_v, z_out.at[pl.ds((w * PK + k) * CH, CH)])

  return pl.kernel(body, out_type=out_type, mesh=_mesh,
                   compiler_params=_sc_params,
                   scratch_types=scratch)(h, sidx, didx)


def _tc_combine(aggp, degp, h, Wn, Ws, b, relu):
  BN = 2000

  def body(a_ref, d_ref, h_ref, wn_ref, ws_ref, b_ref, o_ref):
    agg = a_ref[0] + a_ref[1]
    deg = jnp.maximum(d_ref[0, :, :1] + d_ref[1, :, :1], 1.0)
    neigh = agg / deg
    out = (jnp.dot(neigh, wn_ref[...], preferred_element_type=jnp.float32)
           + jnp.dot(h_ref[...], ws_ref[...], preferred_element_type=jnp.float32)
           + b_ref[...])
    if relu:
      out = jnp.maximum(out, 0.0)
    o_ref[...] = out

  return pl.pallas_call(
      body,
      grid=(N // BN,),
      in_specs=[
          pl.BlockSpec((NC, BN, D), lambda i: (0, i, 0)),
          pl.BlockSpec((NC, BN, D), lambda i: (0, i, 0)),
          pl.BlockSpec((BN, D), lambda i: (i, 0)),
          pl.BlockSpec((D, D), lambda i: (0, 0)),
          pl.BlockSpec((D, D), lambda i: (0, 0)),
          pl.BlockSpec((1, D), lambda i: (0, 0)),
      ],
      out_specs=pl.BlockSpec((BN, D), lambda i: (i, 0)),
      out_shape=jax.ShapeDtypeStruct((N, D), jnp.float32),
  )(aggp, degp, h, Wn, Ws, b.reshape(1, D))


def _tc_mlp(z, PW1, pb1, PW2, pb2, PW3, pb3):
  R = z.shape[0]
  BN = max(b for b in (2048, 1792, 1280, 1024, 512) if R % b == 0)

  def body(z_ref, w1_ref, b1_ref, w2_ref, b2_ref, w3_ref, b3_ref, o_ref):
    a = jnp.maximum(
        jnp.dot(z_ref[...], w1_ref[...], preferred_element_type=jnp.float32)
        + b1_ref[...], 0.0)
    a = jnp.maximum(
        jnp.dot(a, w2_ref[...], preferred_element_type=jnp.float32)
        + b2_ref[...], 0.0)
    o_ref[...] = (jnp.dot(a, w3_ref[...], preferred_element_type=jnp.float32)
                  + b3_ref[...])

  return pl.pallas_call(
      body,
      grid=(R // BN,),
      in_specs=[
          pl.BlockSpec((BN, D), lambda i: (i, 0)),
          pl.BlockSpec((D, D), lambda i: (0, 0)),
          pl.BlockSpec((1, D), lambda i: (0, 0)),
          pl.BlockSpec((D, D), lambda i: (0, 0)),
          pl.BlockSpec((1, D), lambda i: (0, 0)),
          pl.BlockSpec((D, 1), lambda i: (0, 0)),
          pl.BlockSpec((1, 1), lambda i: (0, 0)),
      ],
      out_specs=pl.BlockSpec((BN, 1), lambda i: (i, 0)),
      out_shape=jax.ShapeDtypeStruct((R, 1), jnp.float32),
  )(z, PW1, pb1.reshape(1, D), PW2, pb2.reshape(1, D), PW3, pb3.reshape(1, 1))


def _chunk_edges(src, dst, ew):
  pad = EP - E
  srcp = jnp.concatenate([src, jnp.zeros((pad,), jnp.int32)])
  dstp = jnp.concatenate([dst, jnp.full((pad,), N, jnp.int32)])
  ewp = jnp.concatenate([ew, jnp.zeros((pad,), jnp.float32)])

  def arrange(a):
    return a.reshape(K, NT, CH).transpose(1, 0, 2).reshape(NT * K, CH)

  srcs = arrange(srcp)
  dsts = arrange(dstp)
  ww = jnp.broadcast_to(arrange(ewp)[:, :, None], (NT * K, CH, 16))
  return srcs, dsts, ww


def _pad_pairs(a, b):
  pad = PP - P
  ap = jnp.concatenate([a, jnp.zeros((pad,), jnp.int32)])
  bp = jnp.concatenate([b, jnp.zeros((pad,), jnp.int32)])
  return ap, bp


def kernel(x, edge_index, edge_weight, pos_src, pos_dst, neg_src, neg_dst,
           Wn0, Ws0, b0, Wn1, Ws1, b1, Wn2, Ws2, b2,
           PW1, pb1, PW2, pb2, PW3, pb3):
  src = edge_index[0]
  dst = edge_index[1]
  srcs, dsts, ww = _chunk_edges(src, dst, edge_weight)
  zacc = jnp.zeros((NACC, D), jnp.float32)

  deg = _sc_deg(dsts, zacc)
  agg0 = _sc_agg(x, srcs, dsts, ww, zacc)
  h1 = _tc_combine(agg0, deg, x, Wn0, Ws0, b0, relu=True)
  agg1 = _sc_agg(h1, srcs, dsts, ww, zacc)
  h2 = _tc_combine(agg1, deg, h1, Wn1, Ws1, b1, relu=True)
  agg2 = _sc_agg(h2, srcs, dsts, ww, zacc)
  h3 = _tc_combine(agg2, deg, h2, Wn2, Ws2, b2, relu=False)

  ps, pd = _pad_pairs(pos_src, pos_dst)
  ns, nd = _pad_pairs(neg_src, neg_dst)
  sidx = jnp.concatenate([ps, ns]).reshape(NT, PK, CH)
  didx = jnp.concatenate([pd, nd]).reshape(NT, PK, CH)
  z = _sc_pairs(h3, sidx, didx)
  scores = _tc_mlp(z, PW1, pb1, PW2, pb2, PW3, pb3)
  h_pos = scores[:P]
  h_neg = scores[PP:PP + P]
  return (h_pos, h_neg, h3)


# speedup vs baseline: 4.6477x; 1.0038x over previous
import dataclasses

import jax
import jax.numpy as jnp
from jax import lax
from jax.experimental import pallas as pl
from jax.experimental.pallas import tpu as pltpu
from jax.experimental.pallas import tpu_sc as plsc

N = 10000
E = 320000
P = 10000
D = 128

NC = 2
NS = 16
NT = NC * NS
CH = 88
K = 114
EP = NT * K * CH
NACC = 10112
RPT = NACC // NS

PP = 11264
PK = (2 * PP) // (NT * CH)

_mesh = plsc.VectorSubcoreMesh(core_axis_name="c", subcore_axis_name="s",
                               num_cores=NC, num_subcores=NS)

_sc_params = pltpu.CompilerParams()
if "needs_layout_passes" in pltpu.CompilerParams.__dataclass_fields__:
  _sc_params = dataclasses.replace(_sc_params, needs_layout_passes=False)


def _sc_agg(h, srcs, dsts, ww, zacc):
  out_type = jax.ShapeDtypeStruct((NC, NACC, D), jnp.float32)
  scratch = [
      pltpu.VMEM((CH,), jnp.int32),
      pltpu.VMEM((CH,), jnp.int32),
      pltpu.VMEM((CH,), jnp.int32),
      pltpu.VMEM((CH,), jnp.int32),
      pltpu.VMEM((CH, 16), jnp.float32),
      pltpu.VMEM((CH, 16), jnp.float32),
      pltpu.VMEM((2, CH, D), jnp.float32),
      pltpu.VMEM_SHARED((NACC, D), jnp.float32),
      pltpu.SemaphoreType.DMA,
      pltpu.SemaphoreType.DMA,
      pltpu.SemaphoreType.DMA,
      pltpu.SemaphoreType.DMA,
  ]

  def body(h_hbm, s_hbm, d_hbm, w_hbm, zacc_hbm, agg_out,
           si0, si1, di0, di1, wwv0, wwv1, rows_v, acc_s, g0, g1, m0, m1):
    c = lax.axis_index("c")
    s = lax.axis_index("s")
    w = s * NC + c
    base = w * K
    row0 = s * RPT
    si = [si0, si1]
    di = [di0, di1]
    wwv = [wwv0, wwv1]
    gsem = [g0, g1]
    msem = [m0, m1]

    def meta_copy(g, b):
      pltpu.async_copy(s_hbm.at[base + g], si[b], msem[b])
      pltpu.async_copy(d_hbm.at[base + g], di[b], msem[b])
      pltpu.async_copy(w_hbm.at[base + g], wwv[b], msem[b])

    def meta_wait(g, b):
      pltpu.make_async_copy(s_hbm.at[base + g], si[b], msem[b]).wait()
      pltpu.make_async_copy(d_hbm.at[base + g], di[b], msem[b]).wait()
      pltpu.make_async_copy(w_hbm.at[base + g], wwv[b], msem[b]).wait()

    def gather(g, b):
      return pltpu.async_copy(h_hbm.at[si[b]], rows_v.at[b], gsem[b])

    def scale(b):
      buf = rows_v.at[b]
      wbuf = wwv[b]
      @pl.loop(0, CH)
      def _(e):
        we = wbuf[e, pl.ds(0, 16)]
        for cp in range(D // 16):
          sl = pl.ds(cp * 16, 16)
          buf[e, sl] = buf[e, sl] * we

    def scatter(b, ssem):
      return pltpu.async_copy(rows_v.at[b], acc_s.at[di[b]], ssem, add=True)

    pltpu.sync_copy(zacc_hbm.at[pl.ds(row0, RPT)], acc_s.at[pl.ds(row0, RPT)])

    meta_copy(0, 0)
    meta_copy(1, 1)
    plsc.subcore_barrier()

    @pl.loop(0, K, step=2)
    def _(g):
      meta_wait(g, 0)
      ga = gather(g, 0)
      meta_wait(g + 1, 1)
      gb = gather(g + 1, 1)
      ga.wait()
      scale(0)
      sa = scatter(0, gsem[0])
      @pl.when(g + 2 < K)
      def _():
        pltpu.async_copy(s_hbm.at[base + g + 2], si[0], msem[0])
        pltpu.async_copy(w_hbm.at[base + g + 2], wwv[0], msem[0])
      gb.wait()
      scale(1)
      sb = scatter(1, gsem[1])
      @pl.when(g + 3 < K)
      def _():
        pltpu.async_copy(s_hbm.at[base + g + 3], si[1], msem[1])
        pltpu.async_copy(w_hbm.at[base + g + 3], wwv[1], msem[1])
      sa.wait()
      @pl.when(g + 2 < K)
      def _():
        pltpu.async_copy(d_hbm.at[base + g + 2], di[0], msem[0])
      sb.wait()
      @pl.when(g + 3 < K)
      def _():
        pltpu.async_copy(d_hbm.at[base + g + 3], di[1], msem[1])

    plsc.subcore_barrier()
    pltpu.sync_copy(acc_s.at[pl.ds(row0, RPT)], agg_out.at[c, pl.ds(row0, RPT)])

  return pl.kernel(body, out_type=out_type, mesh=_mesh,
                   compiler_params=_sc_params,
                   scratch_types=scratch)(h, srcs, dsts, ww, zacc)


def _sc_deg(dsts, zacc):
  out_type = jax.ShapeDtypeStruct((NC, NACC, D), jnp.float32)
  scratch = [
      pltpu.VMEM((CH,), jnp.int32),
      pltpu.VMEM((CH,), jnp.int32),
      pltpu.VMEM((CH, D), jnp.float32),
      pltpu.VMEM_SHARED((NACC, D), jnp.float32),
      pltpu.SemaphoreType.DMA,
      pltpu.SemaphoreType.DMA,
      pltpu.SemaphoreType.DMA,
      pltpu.SemaphoreType.DMA,
  ]

  def body(d_hbm, zdeg_hbm, deg_out, b0, b1, ones_v, deg_s, isem0, isem1,
           ssem0, ssem1):
    c = lax.axis_index("c")
    s = lax.axis_index("s")
    w = s * NC + c
    base = w * K
    row0 = s * RPT
    pltpu.sync_copy(zdeg_hbm.at[pl.ds(row0, RPT)], deg_s.at[pl.ds(row0, RPT)])
    @pl.loop(0, CH)
    def _(r):
      for cp in range(D // 16):
        ones_v[r, pl.ds(cp * 16, 16)] = jnp.ones((16,), jnp.float32)
    pltpu.async_copy(d_hbm.at[base], b0, isem0)
    pltpu.async_copy(d_hbm.at[base + 1], b1, isem1)
    plsc.subcore_barrier()
    @pl.loop(0, K, step=2)
    def _(g):
      pltpu.make_async_copy(d_hbm.at[base + g], b0, isem0).wait()
      d0 = pltpu.async_copy(ones_v, deg_s.at[b0], ssem0, add=True)
      pltpu.make_async_copy(d_hbm.at[base + g + 1], b1, isem1).wait()
      d1 = pltpu.async_copy(ones_v, deg_s.at[b1], ssem1, add=True)
      d0.wait()
      @pl.when(g + 2 < K)
      def _():
        pltpu.async_copy(d_hbm.at[base + g + 2], b0, isem0)
      d1.wait()
      @pl.when(g + 3 < K)
      def _():
        pltpu.async_copy(d_hbm.at[base + g + 3], b1, isem1)
    plsc.subcore_barrier()
    pltpu.sync_copy(deg_s.at[pl.ds(row0, RPT)], deg_out.at[c, pl.ds(row0, RPT)])

  return pl.kernel(body, out_type=out_type, mesh=_mesh,
                   compiler_params=_sc_params,
                   scratch_types=scratch)(dsts, zacc)


def _sc_pairs(h, sidx, didx):
  out_type = jax.ShapeDtypeStruct((2 * PP, D), jnp.float32)
  scratch = [
      pltpu.VMEM((PK, CH), jnp.int32),
      pltpu.VMEM((PK, CH), jnp.int32),
      pltpu.VMEM((CH, D), jnp.float32),
      pltpu.VMEM((CH, D), jnp.float32),
      pltpu.SemaphoreType.DMA,
      pltpu.SemaphoreType.DMA,
  ]

  def body(h_hbm, s_hbm, d_hbm, z_out, s_v, d_v, hs_v, hd_v, sem0, sem1):
    c = lax.axis_index("c")
    s = lax.axis_index("s")
    w = s * NC + c
    pltpu.sync_copy(s_hbm.at[w], s_v)
    pltpu.sync_copy(d_hbm.at[w], d_v)
    for k in range(PK):
      da = pltpu.async_copy(h_hbm.at[s_v.at[k]], hs_v, sem0)
      db = pltpu.async_copy(h_hbm.at[d_v.at[k]], hd_v, sem1)
      da.wait()
      db.wait()
      @pl.loop(0, CH)
      def _(r):
        for cp in range(D // 16):
          sl = pl.ds(cp * 16, 16)
          hs_v[r, sl] = hs_v[r, sl] * hd_v[r, sl]
      pltpu.sync_copy(hs_v, z_out.at[pl.ds((w * PK + k) * CH, CH)])

  return pl.kernel(body, out_type=out_type, mesh=_mesh,
                   compiler_params=_sc_params,
                   scratch_types=scratch)(h, sidx, didx)


def _tc_combine(aggp, degp, h, Wn, Ws, b, relu):
  BN = 2000

  def body(a_ref, d_ref, h_ref, wn_ref, ws_ref, b_ref, o_ref):
    agg = a_ref[0] + a_ref[1]
    deg = jnp.maximum(d_ref[0, :, :1] + d_ref[1, :, :1], 1.0)
    neigh = agg / deg
    out = (jnp.dot(neigh, wn_ref[...], preferred_element_type=jnp.float32)
           + jnp.dot(h_ref[...], ws_ref[...], preferred_element_type=jnp.float32)
           + b_ref[...])
    if relu:
      out = jnp.maximum(out, 0.0)
    o_ref[...] = out

  return pl.pallas_call(
      body,
      grid=(N // BN,),
      in_specs=[
          pl.BlockSpec((NC, BN, D), lambda i: (0, i, 0)),
          pl.BlockSpec((NC, BN, D), lambda i: (0, i, 0)),
          pl.BlockSpec((BN, D), lambda i: (i, 0)),
          pl.BlockSpec((D, D), lambda i: (0, 0)),
          pl.BlockSpec((D, D), lambda i: (0, 0)),
          pl.BlockSpec((1, D), lambda i: (0, 0)),
      ],
      out_specs=pl.BlockSpec((BN, D), lambda i: (i, 0)),
      out_shape=jax.ShapeDtypeStruct((N, D), jnp.float32),
  )(aggp, degp, h, Wn, Ws, b.reshape(1, D))


def _tc_mlp(z, PW1, pb1, PW2, pb2, PW3, pb3):
  R = z.shape[0]
  BN = max(b for b in (2048, 1792, 1280, 1024, 512) if R % b == 0)

  def body(z_ref, w1_ref, b1_ref, w2_ref, b2_ref, w3_ref, b3_ref, o_ref):
    a = jnp.maximum(
        jnp.dot(z_ref[...], w1_ref[...], preferred_element_type=jnp.float32)
        + b1_ref[...], 0.0)
    a = jnp.maximum(
        jnp.dot(a, w2_ref[...], preferred_element_type=jnp.float32)
        + b2_ref[...], 0.0)
    o_ref[...] = (jnp.dot(a, w3_ref[...], preferred_element_type=jnp.float32)
                  + b3_ref[...])

  return pl.pallas_call(
      body,
      grid=(R // BN,),
      in_specs=[
          pl.BlockSpec((BN, D), lambda i: (i, 0)),
          pl.BlockSpec((D, D), lambda i: (0, 0)),
          pl.BlockSpec((1, D), lambda i: (0, 0)),
          pl.BlockSpec((D, D), lambda i: (0, 0)),
          pl.BlockSpec((1, D), lambda i: (0, 0)),
          pl.BlockSpec((D, 1), lambda i: (0, 0)),
          pl.BlockSpec((1, 1), lambda i: (0, 0)),
      ],
      out_specs=pl.BlockSpec((BN, 1), lambda i: (i, 0)),
      out_shape=jax.ShapeDtypeStruct((R, 1), jnp.float32),
  )(z, PW1, pb1.reshape(1, D), PW2, pb2.reshape(1, D), PW3, pb3.reshape(1, 1))


def _chunk_edges(src, dst, ew):
  pad = EP - E
  srcp = jnp.concatenate([src, jnp.zeros((pad,), jnp.int32)])
  dstp = jnp.concatenate([dst, jnp.full((pad,), N, jnp.int32)])
  ewp = jnp.concatenate([ew, jnp.zeros((pad,), jnp.float32)])

  def arrange(a):
    return a.reshape(K, NT, CH).transpose(1, 0, 2).reshape(NT * K, CH)

  srcs = arrange(srcp)
  dsts = arrange(dstp)
  ww = jnp.broadcast_to(arrange(ewp)[:, :, None], (NT * K, CH, 16))
  return srcs, dsts, ww


def _pad_pairs(a, b):
  pad = PP - P
  ap = jnp.concatenate([a, jnp.zeros((pad,), jnp.int32)])
  bp = jnp.concatenate([b, jnp.zeros((pad,), jnp.int32)])
  return ap, bp


def kernel(x, edge_index, edge_weight, pos_src, pos_dst, neg_src, neg_dst,
           Wn0, Ws0, b0, Wn1, Ws1, b1, Wn2, Ws2, b2,
           PW1, pb1, PW2, pb2, PW3, pb3):
  src = edge_index[0]
  dst = edge_index[1]
  srcs, dsts, ww = _chunk_edges(src, dst, edge_weight)
  zacc = jnp.zeros((NACC, D), jnp.float32)

  deg = _sc_deg(dsts, zacc)
  agg0 = _sc_agg(x, srcs, dsts, ww, zacc)
  h1 = _tc_combine(agg0, deg, x, Wn0, Ws0, b0, relu=True)
  agg1 = _sc_agg(h1, srcs, dsts, ww, zacc)
  h2 = _tc_combine(agg1, deg, h1, Wn1, Ws1, b1, relu=True)
  agg2 = _sc_agg(h2, srcs, dsts, ww, zacc)
  h3 = _tc_combine(agg2, deg, h2, Wn2, Ws2, b2, relu=False)

  ps, pd = _pad_pairs(pos_src, pos_dst)
  ns, nd = _pad_pairs(neg_src, neg_dst)
  sidx = jnp.concatenate([ps, ns]).reshape(NT, PK, CH)
  didx = jnp.concatenate([pd, nd]).reshape(NT, PK, CH)
  z = _sc_pairs(h3, sidx, didx)
  scores = _tc_mlp(z, PW1, pb1, PW2, pb2, PW3, pb3)
  h_pos = scores[:P]
  h_neg = scores[PP:PP + P]
  return (h_pos, h_neg, h3)
